# baseline (speedup 1.0000x reference)
import jax
import jax.numpy as jnp
from jax import lax
from jax.experimental import pallas as pl
from jax.experimental.pallas import tpu as pltpu
from jax.experimental.pallas import tpu_sc as plsc

M = 2048
C = 128
NC, NS = 2, 16
MP = M + 16
DEPTH = 4


def _sc_segsum(u_flat, gidx, sidx, n_chunks, d):
    def body(u_hbm, gidx_hbm, sidx_hbm, out_hbm,
             acc, ig, isA, gb0, gb1, gb2, gb3, zrow, sem0, sem1, sem2, sem3):
        c = lax.axis_index("c")
        s = lax.axis_index("s")
        bufs = [gb0, gb1, gb2, gb3]
        sems = [sem0, sem1, sem2, sem3]

        z = jnp.zeros((16,), jnp.float32)
        for i in range(16):
            for k in range(d // 16):
                zrow[i, pl.ds(k * 16, 16)] = z
        rows_per_tile = M // NS

        for j in range(2):
            for r in range(rows_per_tile // 16):
                pltpu.sync_copy(
                    zrow, acc.at[pl.ds(j * MP + s * rows_per_tile + r * 16, 16)])

            @pl.when(s == NS - 1)
            def _zero_dummy():
                pltpu.sync_copy(zrow, acc.at[pl.ds(j * MP + M, 16)])

        pltpu.sync_copy(gidx_hbm.at[c, s], ig)
        pltpu.sync_copy(sidx_hbm.at[c, s], isA)

        plsc.subcore_barrier()

        def start(g, gb, sem):
            pltpu.async_copy(u_hbm.at[ig.at[g]], gb, sem)

        def finish(g, gb, sem):
            pltpu.make_async_copy(u_hbm.at[ig.at[g]], gb, sem).wait()
            pltpu.sync_copy(gb, acc.at[isA.at[g]], add=True)

        for k in range(DEPTH):
            start(k, bufs[k], sems[k])

        def outer(u, carry):
            g0 = DEPTH * u
            for k in range(DEPTH):
                finish(g0 + k, bufs[k], sems[k])

                @pl.when(g0 + k + DEPTH < n_chunks)
                def _():
                    start(g0 + k + DEPTH, bufs[k], sems[k])

            return carry

        lax.fori_loop(0, n_chunks // DEPTH, outer, 0)

        plsc.subcore_barrier()

        for j in range(2):
            pltpu.sync_copy(
                acc.at[pl.ds(j * MP + s * rows_per_tile, rows_per_tile)], gb0)
            pltpu.sync_copy(
                gb0, out_hbm.at[c, j, pl.ds(s * rows_per_tile, rows_per_tile)])

    fn = pl.kernel(
        body,
        out_type=jax.ShapeDtypeStruct((NC, 2, M, d), jnp.float32),
        mesh=plsc.VectorSubcoreMesh(core_axis_name="c", subcore_axis_name="s"),
        scratch_types=[
            pltpu.VMEM_SHARED((2 * MP, d), jnp.float32),
            pltpu.VMEM((n_chunks, C), jnp.int32),
            pltpu.VMEM((n_chunks, C), jnp.int32),
            pltpu.VMEM((C, d), jnp.float32),
            pltpu.VMEM((C, d), jnp.float32),
            pltpu.VMEM((C, d), jnp.float32),
            pltpu.VMEM((C, d), jnp.float32),
            pltpu.VMEM((16, d), jnp.float32),
            pltpu.SemaphoreType.DMA,
            pltpu.SemaphoreType.DMA,
            pltpu.SemaphoreType.DMA,
            pltpu.SemaphoreType.DMA,
        ],
    )
    return fn(u_flat, gidx, sidx)


def kernel(Uold, src, dst):
    b, n, d = Uold.shape
    e = src.shape[0]

    gran = 2 * NS * C
    e_pad = ((e + gran - 1) // gran) * gran
    nc_b = e_pad // (NS * C)
    n_chunks = 2 * nc_b
    pad = e_pad - e
    src_p = jnp.concatenate([src, jnp.zeros((pad,), jnp.int32)])
    dst_p = jnp.concatenate([dst, jnp.full((pad,), M, jnp.int32)])

    gs = []
    ss = []
    for c in range(NC):
        gb_ = [(src_p + (2 * c + j) * n).reshape(NS, nc_b, C) for j in range(2)]
        sb_ = [(dst_p + j * MP).reshape(NS, nc_b, C) for j in range(2)]
        gs.append(jnp.concatenate(gb_, axis=1))
        ss.append(jnp.concatenate(sb_, axis=1))
    gidx = jnp.stack(gs)
    sidx = jnp.stack(ss)

    u_flat = Uold.reshape(b * n, d)
    out4 = _sc_segsum(u_flat, gidx, sidx, n_chunks, d)
    return out4.reshape(b, M, d)


# speedup vs baseline: 2.1829x; 2.1829x over previous
import jax
import jax.numpy as jnp
from jax import lax
from jax.experimental import pallas as pl
from jax.experimental.pallas import tpu as pltpu
from jax.experimental.pallas import tpu_sc as plsc

M = 2048
C = 56
NC, NS = 2, 16
MP = M + 16
DEPTH = 4


def _sc_segsum(Uold, gidx, sidx, n_chunks, n, d):
    def body(u_hbm, gidx_hbm, sidx_hbm, zeros_hbm, out_hbm,
             tab, acc, ig, isA, gb0, gb1, gb2, gb3,
             gs0, gs1, gs2, gs3, ss0, ss1, ss2, ss3):
        c = lax.axis_index("c")
        s = lax.axis_index("s")
        bufs = [gb0, gb1, gb2, gb3]
        gsems = [gs0, gs1, gs2, gs3]
        ssems = [ss0, ss1, ss2, ss3]

        rows_per_tile = M // NS

        pltpu.sync_copy(gidx_hbm.at[s], ig)
        pltpu.sync_copy(sidx_hbm.at[s], isA)

        tab_rows_per_tile = n // NS

        for j in range(2):
            pltpu.sync_copy(zeros_hbm, gb1)
            off = 0
            while off < rows_per_tile:
                blk = min(C, rows_per_tile - off)
                pltpu.sync_copy(gb1.at[pl.ds(0, blk)],
                                acc.at[pl.ds(s * rows_per_tile + off, blk)])
                off += blk

            @pl.when(s == NS - 1)
            def _zero_dummy():
                pltpu.sync_copy(gb1.at[pl.ds(0, 16)], acc.at[pl.ds(M, 16)])

            off = 0
            while off < tab_rows_per_tile:
                blk = min(C, tab_rows_per_tile - off)
                row0 = s * tab_rows_per_tile + off
                pltpu.sync_copy(u_hbm.at[2 * c + j, pl.ds(row0, blk)],
                                gb0.at[pl.ds(0, blk)])
                pltpu.sync_copy(gb0.at[pl.ds(0, blk)], tab.at[pl.ds(row0, blk)])
                off += blk

            plsc.subcore_barrier()

            def start_gather(g, b):
                pltpu.async_copy(tab.at[ig.at[g]], bufs[b], gsems[b])

            def wait_gather(g, b):
                pltpu.make_async_copy(tab.at[ig.at[g]], bufs[b], gsems[b]).wait()

            def start_scatter(g, b):
                pltpu.async_copy(bufs[b], acc.at[isA.at[g]], ssems[b], add=True)

            def wait_scatter(g, b):
                pltpu.make_async_copy(
                    bufs[b], acc.at[isA.at[g]], ssems[b]).wait()

            start_gather(0, 0)
            start_gather(1, 1)
            for g in range(4):
                wait_gather(g, g % 4)
                start_scatter(g, g % 4)
                h = g + 2
                if h >= 4:
                    wait_scatter(h - 4, h % 4)
                start_gather(h, h % 4)

            def outer(u, carry):
                g0 = 4 * u + 4
                for k in range(4):
                    g = g0 + k
                    wait_gather(g, k)
                    start_scatter(g, k)
                    hk = (k + 2) % 4

                    @pl.when(g + 2 < n_chunks)
                    def _():
                        wait_scatter(g - 2, hk)
                        start_gather(g + 2, hk)

                return carry

            lax.fori_loop(0, (n_chunks - 4) // 4, outer, 0)

            for k in range(4):
                wait_scatter(n_chunks - 4 + k, k)

            plsc.subcore_barrier()

            off = 0
            while off < rows_per_tile:
                blk = min(C, rows_per_tile - off)
                row0 = s * rows_per_tile + off
                pltpu.sync_copy(acc.at[pl.ds(row0, blk)], gb0.at[pl.ds(0, blk)])
                pltpu.sync_copy(gb0.at[pl.ds(0, blk)],
                                out_hbm.at[c, j, pl.ds(row0, blk)])
                off += blk

    fn = pl.kernel(
        body,
        out_type=jax.ShapeDtypeStruct((NC, 2, M, d), jnp.float32),
        mesh=plsc.VectorSubcoreMesh(core_axis_name="c", subcore_axis_name="s"),
        scratch_types=[
            pltpu.VMEM_SHARED((n, d), jnp.float32),
            pltpu.VMEM_SHARED((MP, d), jnp.float32),
            pltpu.VMEM((n_chunks, C), jnp.int32),
            pltpu.VMEM((n_chunks, C), jnp.int32),
            pltpu.VMEM((C, d), jnp.float32),
            pltpu.VMEM((C, d), jnp.float32),
            pltpu.VMEM((C, d), jnp.float32),
            pltpu.VMEM((C, d), jnp.float32),
            pltpu.SemaphoreType.DMA,
            pltpu.SemaphoreType.DMA,
            pltpu.SemaphoreType.DMA,
            pltpu.SemaphoreType.DMA,
            pltpu.SemaphoreType.DMA,
            pltpu.SemaphoreType.DMA,
            pltpu.SemaphoreType.DMA,
            pltpu.SemaphoreType.DMA,
        ],
    )
    return fn(Uold, gidx, sidx, jnp.zeros((C, d), jnp.float32))


def kernel(Uold, src, dst):
    b, n, d = Uold.shape
    e = src.shape[0]

    gran = 4 * NS * C
    e_pad = ((e + gran - 1) // gran) * gran
    n_chunks = e_pad // (NS * C)
    pad = e_pad - e
    src_p = jnp.concatenate([src, jnp.zeros((pad,), jnp.int32)])
    dst_p = jnp.concatenate([dst, jnp.full((pad,), M, jnp.int32)])

    gidx = src_p.reshape(NS, n_chunks, C)
    sidx = dst_p.reshape(NS, n_chunks, C)

    out4 = _sc_segsum(Uold, gidx, sidx, n_chunks, n, d)
    return out4.reshape(b, M, d)


# speedup vs baseline: 2.4310x; 1.1137x over previous
import jax
import jax.numpy as jnp
from jax import lax
from jax.experimental import pallas as pl
from jax.experimental.pallas import tpu as pltpu
from jax.experimental.pallas import tpu_sc as plsc

M = 2048
C = 56
NC, NS = 2, 16
MP = M + 16
DEPTH = 4


def _sc_segsum(Uold, gidx, sidx, n_chunks, n, d):
    def body(u_hbm, gidx_hbm, sidx_hbm, zeros_hbm, out_hbm,
             tab, acc, ig, isA, gb0, gb1, gb2, gb3,
             gs0, gs1, gs2, gs3, ss0, ss1, ss2, ss3):
        c = lax.axis_index("c")
        s = lax.axis_index("s")
        bufs = [gb0, gb1, gb2, gb3]
        gsems = [gs0, gs1, gs2, gs3]
        ssems = [ss0, ss1, ss2, ss3]

        rows_per_tile = M // NS

        pltpu.sync_copy(gidx_hbm.at[s], ig)
        pltpu.sync_copy(sidx_hbm.at[s], isA)

        tab_rows_per_tile = n // NS

        for j in range(2):
            pltpu.sync_copy(zeros_hbm, gb1)
            off = 0
            while off < rows_per_tile:
                blk = min(C, rows_per_tile - off)
                pltpu.sync_copy(gb1.at[pl.ds(0, blk)],
                                acc.at[pl.ds(s * rows_per_tile + off, blk)])
                off += blk

            @pl.when(s == NS - 1)
            def _zero_dummy():
                pltpu.sync_copy(gb1.at[pl.ds(0, 16)], acc.at[pl.ds(M, 16)])

            row0 = s * tab_rows_per_tile
            pltpu.sync_copy(u_hbm.at[2 * c + j, pl.ds(row0, tab_rows_per_tile)],
                            tab.at[pl.ds(row0, tab_rows_per_tile)])

            plsc.subcore_barrier()

            def start_gather(g, b):
                pltpu.async_copy(tab.at[ig.at[g]], bufs[b], gsems[b])

            def wait_gather(g, b):
                pltpu.make_async_copy(tab.at[ig.at[g]], bufs[b], gsems[b]).wait()

            def start_scatter(g, b):
                pltpu.async_copy(bufs[b], acc.at[isA.at[g]], ssems[b], add=True)

            def wait_scatter(g, b):
                pltpu.make_async_copy(
                    bufs[b], acc.at[isA.at[g]], ssems[b]).wait()

            start_gather(0, 0)
            start_gather(1, 1)
            for g in range(4):
                wait_gather(g, g % 4)
                start_scatter(g, g % 4)
                h = g + 2
                if h >= 4:
                    wait_scatter(h - 4, h % 4)
                start_gather(h, h % 4)

            def outer(u, carry):
                g0 = 4 * u + 4
                for k in range(4):
                    g = g0 + k
                    wait_gather(g, k)
                    start_scatter(g, k)
                    hk = (k + 2) % 4

                    @pl.when(g + 2 < n_chunks)
                    def _():
                        wait_scatter(g - 2, hk)
                        start_gather(g + 2, hk)

                return carry

            lax.fori_loop(0, (n_chunks - 4) // 4, outer, 0)

            for k in range(4):
                wait_scatter(n_chunks - 4 + k, k)

            plsc.subcore_barrier()

            row0 = s * rows_per_tile
            pltpu.sync_copy(acc.at[pl.ds(row0, rows_per_tile)],
                            out_hbm.at[c, j, pl.ds(row0, rows_per_tile)])

    fn = pl.kernel(
        body,
        out_type=jax.ShapeDtypeStruct((NC, 2, M, d), jnp.float32),
        mesh=plsc.VectorSubcoreMesh(core_axis_name="c", subcore_axis_name="s"),
        scratch_types=[
            pltpu.VMEM_SHARED((n, d), jnp.float32),
            pltpu.VMEM_SHARED((MP, d), jnp.float32),
            pltpu.VMEM((n_chunks, C), jnp.int32),
            pltpu.VMEM((n_chunks, C), jnp.int32),
            pltpu.VMEM((C, d), jnp.float32),
            pltpu.VMEM((C, d), jnp.float32),
            pltpu.VMEM((C, d), jnp.float32),
            pltpu.VMEM((C, d), jnp.float32),
            pltpu.SemaphoreType.DMA,
            pltpu.SemaphoreType.DMA,
            pltpu.SemaphoreType.DMA,
            pltpu.SemaphoreType.DMA,
            pltpu.SemaphoreType.DMA,
            pltpu.SemaphoreType.DMA,
            pltpu.SemaphoreType.DMA,
            pltpu.SemaphoreType.DMA,
        ],
    )
    return fn(Uold, gidx, sidx, jnp.zeros((C, d), jnp.float32))


def kernel(Uold, src, dst):
    b, n, d = Uold.shape
    e = src.shape[0]

    gran = 4 * NS * C
    e_pad = ((e + gran - 1) // gran) * gran
    n_chunks = e_pad // (NS * C)
    pad = e_pad - e
    src_p = jnp.concatenate([src, jnp.zeros((pad,), jnp.int32)])
    dst_p = jnp.concatenate([dst, jnp.full((pad,), M, jnp.int32)])

    gidx = src_p.reshape(NS, n_chunks, C)
    sidx = dst_p.reshape(NS, n_chunks, C)

    out4 = _sc_segsum(Uold, gidx, sidx, n_chunks, n, d)
    return out4.reshape(b, M, d)
